# baseline (device time: 467011 ns/iter reference)
import functools

import jax
import jax.numpy as jnp
from jax import lax
from jax.experimental import pallas as pl
from jax.experimental.pallas import tpu as pltpu

N_Z = 4
K = 32
ROW_BLOCK = 128
SUB = 64
LANES = 128
ROUNDS = 8
IDX_BITS = 13
IDX_MASK = (1 << IDX_BITS) - 1
KEY_MASK = ~IDX_MASK
INT_MIN = -(2**31)

_DeviceIdType = getattr(pl, "DeviceIdType", None) or pltpu.DeviceIdType
_MESH = _DeviceIdType.MESH
_CompilerParams = getattr(pltpu, "CompilerParams", None) or pltpu.TPUCompilerParams
_sem_signal = getattr(pl, "semaphore_signal", None) or pltpu.semaphore_signal
_sem_wait = getattr(pl, "semaphore_wait", None) or pltpu.semaphore_wait


def _to_key(x_f32):
    i = lax.bitcast_convert_type(x_f32, jnp.int32)
    return jnp.where(i >= 0, i, i ^ 0x7FFFFFFF)


def _from_key(k):
    i = jnp.where(k >= 0, k, k ^ 0x7FFFFFFF)
    return lax.bitcast_convert_type(i, jnp.float32)


def _local_topk_body(x_ref, out_ref, c_ref):
    key = _to_key(x_ref[...])
    sub = lax.broadcasted_iota(jnp.int32, x_ref.shape, 1)
    lane = lax.broadcasted_iota(jnp.int32, x_ref.shape, 2)
    c_ref[...] = (key & KEY_MASK) | (IDX_MASK - (sub * LANES + lane))
    cands = []
    for _ in range(ROUNDS):
        c = c_ref[...]
        sm = jnp.max(c, axis=1, keepdims=True)
        cands.append(sm.reshape(ROW_BLOCK, LANES))
        c_ref[...] = jnp.where(c == sm, INT_MIN, c)
    cand = jnp.concatenate(cands, axis=1)
    maxes = []
    for _ in range(K):
        cm = jnp.max(cand, axis=1, keepdims=True)
        maxes.append(cm)
        cand = jnp.where(cand == cm, INT_MIN, cand)
    out_ref[:, :] = jnp.concatenate(maxes, axis=1) & KEY_MASK


def _gather_merge_body(k_ref, out_ref, gath_ref, send_sems, recv_sems):
    xi = lax.axis_index("x")
    yi = lax.axis_index("y")
    zi = lax.axis_index("z")

    barrier = pltpu.get_barrier_semaphore()
    for dz in range(1, N_Z):
        _sem_signal(
            barrier, inc=1,
            device_id=(xi, yi, lax.rem(zi + dz, N_Z)),
            device_id_type=_MESH,
        )
    _sem_wait(barrier, N_Z - 1)

    for myz in range(N_Z):
        @pl.when(zi == myz)
        def _(myz=myz):
            gath_ref[myz] = k_ref[:, :]
            sends = []
            for dz in range(1, N_Z):
                peer = (myz + dz) % N_Z
                r = pltpu.make_async_remote_copy(
                    src_ref=k_ref,
                    dst_ref=gath_ref.at[myz],
                    send_sem=send_sems.at[dz],
                    recv_sem=recv_sems.at[myz],
                    device_id=(xi, yi, peer),
                    device_id_type=_MESH,
                )
                r.start()
                sends.append(r)
            for src in range(N_Z):
                if src == myz:
                    continue
                pltpu.make_async_remote_copy(
                    src_ref=k_ref,
                    dst_ref=gath_ref.at[src],
                    send_sem=send_sems.at[0],
                    recv_sem=recv_sems.at[src],
                    device_id=(xi, yi, myz),
                    device_id_type=_MESH,
                ).wait_recv()
            for r in sends:
                r.wait_send()

    g = gath_ref[:, :, :]
    src_id = lax.broadcasted_iota(jnp.int32, g.shape, 0)
    col_id = lax.broadcasted_iota(jnp.int32, g.shape, 2)
    c = (g & KEY_MASK) | (IDX_MASK - (src_id * K + col_id))
    maxes = []
    for _ in range(K):
        cm = jnp.max(jnp.max(c, axis=0), axis=1, keepdims=True)
        maxes.append(cm)
        c = jnp.where(c == cm[None, :, :], INT_MIN, c)
    keys = jnp.concatenate(maxes, axis=1) & KEY_MASK
    out_ref[:, :] = _from_key(keys)

    @functools.partial(pl.run_scoped, sem2=pltpu.SemaphoreType.REGULAR)
    def _(sem2):
        for dz in range(1, N_Z):
            _sem_signal(
                sem2, inc=1,
                device_id=(xi, yi, lax.rem(zi + dz, N_Z)),
                device_id_type=_MESH,
            )
        _sem_wait(sem2, N_Z - 1)


def kernel(x):
    m, n = x.shape
    assert n == IDX_MASK + 1, (m, n)
    assert m % ROW_BLOCK == 0

    local_keys = pl.pallas_call(
        _local_topk_body,
        grid=(m // ROW_BLOCK,),
        in_specs=[pl.BlockSpec((ROW_BLOCK, SUB, LANES), lambda i: (i, 0, 0))],
        out_specs=pl.BlockSpec((ROW_BLOCK, K), lambda i: (i, 0)),
        out_shape=jax.ShapeDtypeStruct((m, K), jnp.int32),
        scratch_shapes=[pltpu.VMEM((ROW_BLOCK, SUB, LANES), jnp.int32)],
    )(x.reshape(m, SUB, LANES))

    return pl.pallas_call(
        _gather_merge_body,
        out_shape=jax.ShapeDtypeStruct((m, K), jnp.float32),
        in_specs=[pl.BlockSpec(memory_space=pltpu.VMEM)],
        out_specs=pl.BlockSpec(memory_space=pltpu.VMEM),
        scratch_shapes=[
            pltpu.VMEM((N_Z, m, K), jnp.int32),
            pltpu.SemaphoreType.DMA((N_Z,)),
            pltpu.SemaphoreType.DMA((N_Z,)),
        ],
        compiler_params=_CompilerParams(collective_id=0),
    )(local_keys)


# device time: 133856 ns/iter; 3.4889x vs baseline; 3.4889x over previous
import functools

import jax
import jax.numpy as jnp
from jax import lax
from jax.experimental import pallas as pl
from jax.experimental.pallas import tpu as pltpu

N_Z = 4
K = 32
ROW_BLOCK = 128
SUB = 64
LANES = 128
ROUNDS = 8
IDX_BITS = 13
IDX_MASK = (1 << IDX_BITS) - 1
KEY_MASK = ~IDX_MASK
INT_MIN = -(2**31)

_DeviceIdType = getattr(pl, "DeviceIdType", None) or pltpu.DeviceIdType
_MESH = _DeviceIdType.MESH
_CompilerParams = getattr(pltpu, "CompilerParams", None) or pltpu.TPUCompilerParams
_sem_signal = getattr(pl, "semaphore_signal", None) or pltpu.semaphore_signal
_sem_wait = getattr(pl, "semaphore_wait", None) or pltpu.semaphore_wait


def _to_key(x_f32):
    i = lax.bitcast_convert_type(x_f32, jnp.int32)
    return jnp.where(i >= 0, i, i ^ 0x7FFFFFFF)


def _from_key(k):
    i = jnp.where(k >= 0, k, k ^ 0x7FFFFFFF)
    return lax.bitcast_convert_type(i, jnp.float32)


def _local_topk_body(x_ref, out_ref, c_ref):
    n = x_ref.shape[1]
    key = _to_key(x_ref[:, :])
    col = lax.broadcasted_iota(jnp.int32, x_ref.shape, 1)
    c_ref[:, :] = (key & KEY_MASK) | (IDX_MASK - col)
    cands = []
    for r in range(ROUNDS):
        c = c_ref[:, :]
        t = c
        w = n
        while w > LANES:
            w //= 2
            t = jnp.maximum(t[:, :w], t[:, w:])
        cands.append(t)
        if r < ROUNDS - 1:
            sm = jnp.concatenate([t] * (n // LANES), axis=1)
            c_ref[:, :] = jnp.where(c == sm, INT_MIN, c)
    cand = jnp.concatenate(cands, axis=1)
    maxes = []
    for _ in range(K):
        cm = jnp.max(cand, axis=1, keepdims=True)
        maxes.append(cm)
        cand = jnp.where(cand == cm, INT_MIN, cand)
    out_ref[:, :] = jnp.concatenate(maxes, axis=1) & KEY_MASK


def _gather_merge_body(k_ref, out_ref, gath_ref, send_sems, recv_sems):
    xi = lax.axis_index("x")
    yi = lax.axis_index("y")
    zi = lax.axis_index("z")

    barrier = pltpu.get_barrier_semaphore()
    for dz in range(1, N_Z):
        _sem_signal(
            barrier, inc=1,
            device_id=(xi, yi, lax.rem(zi + dz, N_Z)),
            device_id_type=_MESH,
        )
    _sem_wait(barrier, N_Z - 1)

    for myz in range(N_Z):
        @pl.when(zi == myz)
        def _(myz=myz):
            gath_ref[myz] = k_ref[:, :]
            sends = []
            for dz in range(1, N_Z):
                peer = (myz + dz) % N_Z
                r = pltpu.make_async_remote_copy(
                    src_ref=k_ref,
                    dst_ref=gath_ref.at[myz],
                    send_sem=send_sems.at[dz],
                    recv_sem=recv_sems.at[myz],
                    device_id=(xi, yi, peer),
                    device_id_type=_MESH,
                )
                r.start()
                sends.append(r)
            for src in range(N_Z):
                if src == myz:
                    continue
                pltpu.make_async_remote_copy(
                    src_ref=k_ref,
                    dst_ref=gath_ref.at[src],
                    send_sem=send_sems.at[0],
                    recv_sem=recv_sems.at[src],
                    device_id=(xi, yi, myz),
                    device_id_type=_MESH,
                ).wait_recv()
            for r in sends:
                r.wait_send()

    g = gath_ref[:, :, :]
    src_id = lax.broadcasted_iota(jnp.int32, g.shape, 0)
    col_id = lax.broadcasted_iota(jnp.int32, g.shape, 2)
    c = (g & KEY_MASK) | (IDX_MASK - (src_id * K + col_id))
    maxes = []
    for _ in range(K):
        cm = jnp.max(jnp.max(c, axis=0), axis=1, keepdims=True)
        maxes.append(cm)
        c = jnp.where(c == cm[None, :, :], INT_MIN, c)
    keys = jnp.concatenate(maxes, axis=1) & KEY_MASK
    out_ref[:, :] = _from_key(keys)

    @functools.partial(pl.run_scoped, sem2=pltpu.SemaphoreType.REGULAR)
    def _(sem2):
        for dz in range(1, N_Z):
            _sem_signal(
                sem2, inc=1,
                device_id=(xi, yi, lax.rem(zi + dz, N_Z)),
                device_id_type=_MESH,
            )
        _sem_wait(sem2, N_Z - 1)


def kernel(x):
    m, n = x.shape
    assert n == IDX_MASK + 1, (m, n)
    assert m % ROW_BLOCK == 0

    local_keys = pl.pallas_call(
        _local_topk_body,
        grid=(m // ROW_BLOCK,),
        in_specs=[pl.BlockSpec((ROW_BLOCK, n), lambda i: (i, 0))],
        out_specs=pl.BlockSpec((ROW_BLOCK, K), lambda i: (i, 0)),
        out_shape=jax.ShapeDtypeStruct((m, K), jnp.int32),
        scratch_shapes=[pltpu.VMEM((ROW_BLOCK, n), jnp.int32)],
    )(x)

    return pl.pallas_call(
        _gather_merge_body,
        out_shape=jax.ShapeDtypeStruct((m, K), jnp.float32),
        in_specs=[pl.BlockSpec(memory_space=pltpu.VMEM)],
        out_specs=pl.BlockSpec(memory_space=pltpu.VMEM),
        scratch_shapes=[
            pltpu.VMEM((N_Z, m, K), jnp.int32),
            pltpu.SemaphoreType.DMA((N_Z,)),
            pltpu.SemaphoreType.DMA((N_Z,)),
        ],
        compiler_params=_CompilerParams(collective_id=0),
    )(local_keys)


# device time: 102908 ns/iter; 4.5381x vs baseline; 1.3007x over previous
import functools

import jax
import jax.numpy as jnp
from jax import lax
from jax.experimental import pallas as pl
from jax.experimental.pallas import tpu as pltpu

N_Z = 4
K = 32
ROW_BLOCK = 128
SUB = 64
LANES = 128
ROUNDS = 8
IDX_BITS = 13
IDX_MASK = (1 << IDX_BITS) - 1
KEY_MASK = ~IDX_MASK
INT_MIN = -(2**31)

_DeviceIdType = getattr(pl, "DeviceIdType", None) or pltpu.DeviceIdType
_MESH = _DeviceIdType.MESH
_CompilerParams = getattr(pltpu, "CompilerParams", None) or pltpu.TPUCompilerParams
_sem_signal = getattr(pl, "semaphore_signal", None) or pltpu.semaphore_signal
_sem_wait = getattr(pl, "semaphore_wait", None) or pltpu.semaphore_wait


def _to_key(x_f32):
    i = lax.bitcast_convert_type(x_f32, jnp.int32)
    return jnp.where(i >= 0, i, i ^ 0x7FFFFFFF)


def _from_key(k):
    i = jnp.where(k >= 0, k, k ^ 0x7FFFFFFF)
    return lax.bitcast_convert_type(i, jnp.float32)


def _local_topk_body(x_ref, out_ref, c_ref):
    n = x_ref.shape[1]
    key = _to_key(x_ref[:, :])
    col = lax.broadcasted_iota(jnp.int32, x_ref.shape, 1)
    c_ref[:, :] = (key & KEY_MASK) | (IDX_MASK - col)
    cands = []
    for r in range(ROUNDS):
        c = c_ref[:, :]
        t = c
        w = n
        while w > LANES:
            w //= 2
            t = jnp.maximum(t[:, :w], t[:, w:])
        cands.append(t)
        if r < ROUNDS - 1:
            sm = jnp.concatenate([t] * (n // LANES), axis=1)
            c_ref[:, :] = jnp.where(c == sm, INT_MIN, c)
    cand = jnp.concatenate(cands, axis=1)
    maxes = []
    for _ in range(K):
        cm = jnp.max(cand, axis=1, keepdims=True)
        maxes.append(cm)
        cand = jnp.where(cand == cm, INT_MIN, cand)
    out_ref[:, :] = jnp.concatenate(maxes, axis=1) & KEY_MASK


def _gather_merge_body(k_ref, out_ref, gath_ref, send_sems, recv_sems):
    xi = lax.axis_index("x")
    yi = lax.axis_index("y")
    zi = lax.axis_index("z")

    barrier = pltpu.get_barrier_semaphore()
    for dz in range(1, N_Z):
        _sem_signal(
            barrier, inc=1,
            device_id=(xi, yi, lax.rem(zi + dz, N_Z)),
            device_id_type=_MESH,
        )
    _sem_wait(barrier, N_Z - 1)

    for myz in range(N_Z):
        @pl.when(zi == myz)
        def _(myz=myz):
            gath_ref[myz] = k_ref[:, :]
            sends = []
            for dz in range(1, N_Z):
                peer = (myz + dz) % N_Z
                r = pltpu.make_async_remote_copy(
                    src_ref=k_ref,
                    dst_ref=gath_ref.at[myz],
                    send_sem=send_sems.at[dz],
                    recv_sem=recv_sems.at[myz],
                    device_id=(xi, yi, peer),
                    device_id_type=_MESH,
                )
                r.start()
                sends.append(r)
            for src in range(N_Z):
                if src == myz:
                    continue
                pltpu.make_async_remote_copy(
                    src_ref=k_ref,
                    dst_ref=gath_ref.at[src],
                    send_sem=send_sems.at[0],
                    recv_sem=recv_sems.at[src],
                    device_id=(xi, yi, myz),
                    device_id_type=_MESH,
                ).wait_recv()
            for r in sends:
                r.wait_send()

    g = gath_ref[:, :, :]
    src_id = lax.broadcasted_iota(jnp.int32, g.shape, 0)
    rank_id = lax.broadcasted_iota(jnp.int32, g.shape, 1)
    c = (g & KEY_MASK) | (IDX_MASK - (src_id * K + rank_id))
    maxes = []
    for _ in range(K):
        cm = jnp.max(jnp.max(c, axis=0), axis=0, keepdims=True)
        maxes.append(cm)
        c = jnp.where(c == cm[None, :, :], INT_MIN, c)
    keys = jnp.concatenate(maxes, axis=0) & KEY_MASK
    out_ref[:, :] = _from_key(keys)

    @functools.partial(pl.run_scoped, sem2=pltpu.SemaphoreType.REGULAR)
    def _(sem2):
        for dz in range(1, N_Z):
            _sem_signal(
                sem2, inc=1,
                device_id=(xi, yi, lax.rem(zi + dz, N_Z)),
                device_id_type=_MESH,
            )
        _sem_wait(sem2, N_Z - 1)


def kernel(x):
    m, n = x.shape
    assert n == IDX_MASK + 1, (m, n)
    assert m % ROW_BLOCK == 0

    local_keys = pl.pallas_call(
        _local_topk_body,
        grid=(m // ROW_BLOCK,),
        in_specs=[pl.BlockSpec((ROW_BLOCK, n), lambda i: (i, 0))],
        out_specs=pl.BlockSpec((ROW_BLOCK, K), lambda i: (i, 0)),
        out_shape=jax.ShapeDtypeStruct((m, K), jnp.int32),
        scratch_shapes=[pltpu.VMEM((ROW_BLOCK, n), jnp.int32)],
    )(x)

    out_t = pl.pallas_call(
        _gather_merge_body,
        out_shape=jax.ShapeDtypeStruct((K, m), jnp.float32),
        in_specs=[pl.BlockSpec(memory_space=pltpu.VMEM)],
        out_specs=pl.BlockSpec(memory_space=pltpu.VMEM),
        scratch_shapes=[
            pltpu.VMEM((N_Z, K, m), jnp.int32),
            pltpu.SemaphoreType.DMA((N_Z,)),
            pltpu.SemaphoreType.DMA((N_Z,)),
        ],
        compiler_params=_CompilerParams(collective_id=0),
    )(local_keys.T)
    return out_t.T
